# baseline (speedup 1.0000x reference)
import functools

import jax
import jax.numpy as jnp
from jax import lax
from jax.experimental import pallas as pl
from jax.experimental.pallas import tpu as pltpu
from jax.experimental.pallas import tpu_sc as plsc

B = 8
L = 16384
IN = 64
OUT = 16
K = 4
FLAT = B * L

NC = 2
NS = 16
NW = NC * NS
RW = FLAT // NW
CH = 256
NCH = RW // CH
GRP = CH // 128
LANES = 16


def _mm_body(x_ref, w_ref, b_ref, o0, o1, o2, o3):
    res = (
        jnp.dot(x_ref[:], w_ref[:], preferred_element_type=jnp.float32)
        + b_ref[0:1, :]
    )
    o0[:] = res[:, 0 * OUT:1 * OUT]
    o1[:] = res[:, 1 * OUT:2 * OUT]
    o2[:] = res[:, 2 * OUT:3 * OUT]
    o3[:] = res[:, 3 * OUT:4 * OUT]


def _project(x_flat, w_cat, bvec):
    blk = 2048
    grid = FLAT // blk
    return pl.pallas_call(
        _mm_body,
        grid=(grid,),
        in_specs=[
            pl.BlockSpec((blk, IN), lambda i: (i, 0)),
            pl.BlockSpec((IN, K * OUT), lambda i: (0, 0)),
            pl.BlockSpec((8, K * OUT), lambda i: (0, 0)),
        ],
        out_specs=[pl.BlockSpec((blk, OUT), lambda i: (i, 0))] * K,
        out_shape=[jax.ShapeDtypeStruct((FLAT, OUT), jnp.float32)] * K,
    )(x_flat, w_cat, bvec)


def _sc_body(z0, z1, z2, z3, idxt_hbm, out_hbm, idx_v, gidx_v, buf_v,
             obuf_v, sem0, sem1):
    zs = (z0, z1, z2, z3)
    sems = (sem0, sem1)
    wid = lax.axis_index("s") * NC + lax.axis_index("c")
    batch = wid // (L // RW)
    j0 = (wid % (L // RW)) * RW
    row0 = wid * RW

    pltpu.sync_copy(idxt_hbm.at[:, pl.ds(j0, RW)], idx_v)

    base = batch * L

    def idx_body(i, _):
        off = pl.multiple_of(i * LANES, LANES)
        r128 = i // 8
        lane0 = (i % 8) * LANES
        for k in range(K):
            v = idx_v[k, pl.ds(off, LANES)]
            gidx_v[k * (RW // 128) + r128, pl.ds(lane0, LANES)] = v + base
        return 0

    lax.fori_loop(0, RW // LANES, idx_body, 0)

    def copies(s, c):
        out = []
        for k in range(K):
            for g in range(GRP):
                out.append(pltpu.make_async_copy(
                    zs[k].at[gidx_v.at[k * (RW // 128) + c * GRP + g, :]],
                    buf_v.at[s, k, pl.ds(g * 128, 128), :], sems[s]))
        return out

    def start(s, c):
        for cp in copies(s, c):
            cp.start()

    def finish(s, c):
        for cp in copies(s, c):
            cp.wait()

    def compute_write(s, c):
        def row_body(q, _):
            for u in range(4):
                r = q * 4 + u
                acc = buf_v[s, 0, r, :] + buf_v[s, 1, r, :]
                acc = acc + buf_v[s, 2, r, :]
                acc = acc + buf_v[s, 3, r, :]
                obuf_v[r, :] = jnp.maximum(acc, 0.0)
            return 0

        lax.fori_loop(0, CH // 4, row_body, 0)
        pltpu.sync_copy(obuf_v, out_hbm.at[pl.ds(row0 + c * CH, CH)])

    start(0, 0)

    def chunk_body(cc, _):
        c0 = cc * 2
        start(1, c0 + 1)
        finish(0, c0)
        compute_write(0, c0)

        @pl.when(cc + 1 < NCH // 2)
        def _():
            start(0, c0 + 2)

        finish(1, c0 + 1)
        compute_write(1, c0 + 1)
        return 0

    lax.fori_loop(0, NCH // 2, chunk_body, 0)


@functools.cache
def _sc_gather_reduce():
    return pl.kernel(
        _sc_body,
        out_type=jax.ShapeDtypeStruct((FLAT, OUT), jnp.float32),
        mesh=plsc.VectorSubcoreMesh(
            core_axis_name="c", subcore_axis_name="s", num_cores=NC,
            num_subcores=NS),
        scratch_types=[
            pltpu.VMEM((K, RW), jnp.int32),
            pltpu.VMEM((K * RW // 128, 128), jnp.int32),
            pltpu.VMEM((2, K, CH, OUT), jnp.float32),
            pltpu.VMEM((CH, OUT), jnp.float32),
            pltpu.SemaphoreType.DMA,
            pltpu.SemaphoreType.DMA,
        ],
        compiler_params=pltpu.CompilerParams(use_tc_tiling_on_sc=False),
    )


def kernel(x, mask, bias, index_tensor):
    x_flat = x.reshape(FLAT, IN)
    w_cat = jnp.transpose(mask, (1, 0, 2)).reshape(IN, K * OUT)
    brow = jnp.concatenate(
        [jnp.full((OUT,), bias[-1], jnp.float32),
         jnp.zeros((K * OUT - OUT,), jnp.float32)])
    bvec = jnp.broadcast_to(brow, (8, K * OUT))

    z0, z1, z2, z3 = _project(x_flat, w_cat, bvec)
    idxt = jnp.transpose(index_tensor).astype(jnp.int32)

    out = _sc_gather_reduce()(z0, z1, z2, z3, idxt)
    return out.reshape(B, L, OUT)


# speedup vs baseline: 1.1322x; 1.1322x over previous
import functools

import jax
import jax.numpy as jnp
from jax import lax
from jax.experimental import pallas as pl
from jax.experimental.pallas import tpu as pltpu
from jax.experimental.pallas import tpu_sc as plsc

B = 8
L = 16384
IN = 64
OUT = 16
K = 4
FLAT = B * L

NC = 2
NS = 16
NW = NC * NS
RW = FLAT // NW
CH = 256
NCH = RW // CH
GRP = CH // 128
LANES = 16


def _mm_body(x_ref, w_ref, b_ref, o_ref):
    o_ref[:] = (
        jnp.dot(x_ref[:], w_ref[:], preferred_element_type=jnp.float32)
        + b_ref[0:1, :]
    )


def _project(x_flat, w_cat, bvec):
    blk = 2048
    grid = FLAT // blk
    return pl.pallas_call(
        _mm_body,
        grid=(grid,),
        in_specs=[
            pl.BlockSpec((blk, IN), lambda i: (i, 0)),
            pl.BlockSpec((IN, K * OUT), lambda i: (0, 0)),
            pl.BlockSpec((8, K * OUT), lambda i: (0, 0)),
        ],
        out_specs=pl.BlockSpec((blk, K * OUT), lambda i: (i, 0)),
        out_shape=jax.ShapeDtypeStruct((FLAT, K * OUT), jnp.float32),
    )(x_flat, w_cat, bvec)


def _sc_body(z_hbm, idxt_hbm, out_hbm, idx_v, gidx_v, buf_v,
             obuf_v, sem0, sem1):
    sems = (sem0, sem1)
    wid = lax.axis_index("s") * NC + lax.axis_index("c")
    batch = wid // (L // RW)
    j0 = (wid % (L // RW)) * RW
    row0 = wid * RW

    pltpu.sync_copy(idxt_hbm.at[:, pl.ds(j0, RW)], idx_v)

    base = batch * (L * K)

    def idx_body(i, _):
        off = pl.multiple_of(i * LANES, LANES)
        r128 = i // 8
        lane0 = (i % 8) * LANES
        for k in range(K):
            v = idx_v[k, pl.ds(off, LANES)]
            gidx_v[k * (RW // 128) + r128, pl.ds(lane0, LANES)] = (
                v * K + (base + k))
        return 0

    lax.fori_loop(0, RW // LANES, idx_body, 0)

    def copies(s, c):
        out = []
        for k in range(K):
            for g in range(GRP):
                out.append(pltpu.make_async_copy(
                    z_hbm.at[gidx_v.at[k * (RW // 128) + c * GRP + g, :]],
                    buf_v.at[s, k, pl.ds(g * 128, 128), :], sems[s]))
        return out

    def start(s, c):
        for cp in copies(s, c):
            cp.start()

    def finish(s, c):
        for cp in copies(s, c):
            cp.wait()

    def compute_write(s, c):
        def row_body(q, _):
            for u in range(4):
                r = q * 4 + u
                acc = buf_v[s, 0, r, :] + buf_v[s, 1, r, :]
                acc = acc + buf_v[s, 2, r, :]
                acc = acc + buf_v[s, 3, r, :]
                obuf_v[r, :] = jnp.maximum(acc, 0.0)
            return 0

        lax.fori_loop(0, CH // 4, row_body, 0)
        pltpu.sync_copy(obuf_v, out_hbm.at[pl.ds(row0 + c * CH, CH)])

    start(0, 0)

    def chunk_body(cc, _):
        c0 = cc * 2
        start(1, c0 + 1)
        finish(0, c0)
        compute_write(0, c0)

        @pl.when(cc + 1 < NCH // 2)
        def _():
            start(0, c0 + 2)

        finish(1, c0 + 1)
        compute_write(1, c0 + 1)
        return 0

    lax.fori_loop(0, NCH // 2, chunk_body, 0)


@functools.cache
def _sc_gather_reduce():
    return pl.kernel(
        _sc_body,
        out_type=jax.ShapeDtypeStruct((FLAT, OUT), jnp.float32),
        mesh=plsc.VectorSubcoreMesh(
            core_axis_name="c", subcore_axis_name="s", num_cores=NC,
            num_subcores=NS),
        scratch_types=[
            pltpu.VMEM((K, RW), jnp.int32),
            pltpu.VMEM((K * RW // 128, 128), jnp.int32),
            pltpu.VMEM((2, K, CH, OUT), jnp.float32),
            pltpu.VMEM((CH, OUT), jnp.float32),
            pltpu.SemaphoreType.DMA,
            pltpu.SemaphoreType.DMA,
        ],
        compiler_params=pltpu.CompilerParams(use_tc_tiling_on_sc=False),
    )


def kernel(x, mask, bias, index_tensor):
    x_flat = x.reshape(FLAT, IN)
    w_cat = jnp.transpose(mask, (1, 0, 2)).reshape(IN, K * OUT)
    brow = jnp.concatenate(
        [jnp.full((OUT,), bias[-1], jnp.float32),
         jnp.zeros((K * OUT - OUT,), jnp.float32)])
    bvec = jnp.broadcast_to(brow, (8, K * OUT))

    z = _project(x_flat, w_cat, bvec)
    z_rows = z.reshape(FLAT * K, OUT)
    idxt = jnp.transpose(index_tensor).astype(jnp.int32)

    out = _sc_gather_reduce()(z_rows, idxt)
    return out.reshape(B, L, OUT)
